# baseline (speedup 1.0000x reference)
import functools

import jax
import jax.numpy as jnp
from jax import lax
from jax.experimental import pallas as pl
from jax.experimental.pallas import tpu as pltpu
from jax.experimental.pallas import tpu_sc as plsc

TABLE_SIZE = 100003
NUM_HEADS = 8
EMBED_DIM = 512
HEAD_DIM = EMBED_DIM // NUM_HEADS
TOTAL_TABLE = TABLE_SIZE * NUM_HEADS

_INFO = plsc.get_sparse_core_info()
NC = _INFO.num_cores
NS = _INFO.num_subcores
NW = NC * NS
L = _INFO.num_lanes

CHUNK = 512


def _mod(x):
    xf = x.astype(jnp.float32)
    q = (xf * jnp.float32(1.0 / TOTAL_TABLE)).astype(jnp.int32)
    r = x - q * TOTAL_TABLE
    r = jnp.where(r < 0, r + TOTAL_TABLE, r)
    r = jnp.where(r >= TOTAL_TABLE, r - TOTAL_TABLE, r)
    return r


def _modmul(t, p_hi, p_lo):
    a = _mod(t * p_hi)
    b = _mod(a * 1024)
    return _mod(b + t * p_lo)


@functools.partial(jax.jit, static_argnums=(4, 5))
def _launch(prev_ids, cur_ids, primes_flat, memory_table, n_pos, n_rows):
    p_per_w = n_pos // NW
    r_per_w = p_per_w * NUM_HEADS
    n_chunk = r_per_w // CHUNK
    mesh = plsc.VectorSubcoreMesh(core_axis_name="c", subcore_axis_name="s")

    @functools.partial(
        pl.kernel,
        mesh=mesh,
        compiler_params=pltpu.CompilerParams(
            needs_layout_passes=False, use_tc_tiling_on_sc=False),
        out_type=jax.ShapeDtypeStruct((n_rows, HEAD_DIM), jnp.float32),
        scratch_types=[
            pltpu.VMEM((p_per_w,), jnp.int32),
            pltpu.VMEM((p_per_w,), jnp.int32),
            pltpu.VMEM((64,), jnp.int32),
            pltpu.VMEM((r_per_w,), jnp.int32),
            pltpu.VMEM((CHUNK, HEAD_DIM), jnp.float32),
            pltpu.VMEM((CHUNK, HEAD_DIM), jnp.float32),
            pltpu.SemaphoreType.DMA,
            pltpu.SemaphoreType.DMA,
        ],
    )
    def k(prev_hbm, cur_hbm, primes_hbm, table_hbm, out_hbm,
          prev_v, cur_v, primes_v, idx_v, rows0, rows1, sem_g, sem_s):
        w = lax.axis_index("s") * NC + lax.axis_index("c")
        base_p = w * p_per_w
        pltpu.sync_copy(prev_hbm.at[pl.ds(base_p, p_per_w)], prev_v)
        pltpu.sync_copy(cur_hbm.at[pl.ds(base_p, p_per_w)], cur_v)
        pltpu.sync_copy(primes_hbm, primes_v)

        ph0 = primes_v[pl.ds(0, L)]
        pl0 = primes_v[pl.ds(L, L)]
        ph1 = primes_v[pl.ds(2 * L, L)]
        pl1 = primes_v[pl.ds(3 * L, L)]
        iot = lax.iota(jnp.int32, L)
        lane_p = iot >> 3

        def hash_body(i, carry):
            p = lane_p + jnp.int32(2) * i
            tp = plsc.load_gather(prev_v, [p])
            tc = plsc.load_gather(cur_v, [p])
            h = _modmul(tp, ph0, pl0) + _modmul(tc, ph1, pl1)
            h = jnp.where(h >= TOTAL_TABLE, h - TOTAL_TABLE, h)
            idx_v[pl.ds(i * jnp.int32(L), L)] = h
            return carry

        lax.fori_loop(jnp.int32(0), jnp.int32(r_per_w // L), hash_body,
                      jnp.int32(0))

        row_base = w * r_per_w
        bufs = (rows0, rows1)
        gathers = [None] * n_chunk
        gathers[0] = pltpu.async_copy(
            table_hbm.at[idx_v.at[pl.ds(0, CHUNK)]], bufs[0], sem_g)
        stores = [None] * n_chunk
        for c in range(n_chunk):
            gathers[c].wait()
            if c >= 1:
                stores[c - 1].wait()
            if c + 1 < n_chunk:
                gathers[c + 1] = pltpu.async_copy(
                    table_hbm.at[idx_v.at[pl.ds((c + 1) * CHUNK, CHUNK)]],
                    bufs[(c + 1) % 2], sem_g)
            stores[c] = pltpu.async_copy(
                bufs[c % 2], out_hbm.at[pl.ds(row_base + c * CHUNK, CHUNK)],
                sem_s)
        stores[n_chunk - 1].wait()

    return k(prev_ids, cur_ids, primes_flat, memory_table)


def kernel(input_ids, memory_table, hash_primes):
    b, s = input_ids.shape
    n_pos = b * s
    n_rows = n_pos * NUM_HEADS

    ids32 = input_ids.astype(jnp.int32)
    cur = ids32.reshape(-1)
    prev = jnp.pad(ids32, ((0, 0), (1, 0)))[:, :-1].reshape(-1)

    pm = (hash_primes % TOTAL_TABLE).astype(jnp.int32)
    ph = pm >> 10
    plo = pm & 1023
    primes_flat = jnp.concatenate([
        jnp.tile(ph[:, 0], 2), jnp.tile(plo[:, 0], 2),
        jnp.tile(ph[:, 1], 2), jnp.tile(plo[:, 1], 2),
    ]).astype(jnp.int32)

    out = _launch(prev, cur, primes_flat, memory_table, n_pos, n_rows)
    return out.reshape(b, s, EMBED_DIM)
